# baseline (speedup 1.0000x reference)
import jax
import jax.numpy as jnp
from jax.experimental import pallas as pl
from jax.experimental.pallas import tpu as pltpu

_HI = jax.lax.Precision.HIGHEST
_N = 2048
_FB = 128
_DB = 128
_NDB = _N // _DB
_THRESH = 0.1 * _N


def _lrelu(x):
    return jnp.where(x >= 0, x, 0.01 * x)


def _bnorm(x, g, b, eps=1e-5):
    mu = jnp.mean(x, axis=0, keepdims=True)
    var = jnp.mean((x - mu) * (x - mu), axis=0, keepdims=True)
    return (x - mu) / jnp.sqrt(var + eps) * g + b


def _feat_body(x_ref, m1_ref, b1_ref, m2_ref, b2_ref, fw_ref, fb_ref, h_ref):
    x = x_ref[...]
    xc = jnp.concatenate([x[:, dy:dy + 24, :] for dy in range(5)], axis=2)
    xc = xc.reshape(_FB * 24, 140)
    h1 = jax.lax.dot(xc, m1_ref[...], precision=_HI) + b1_ref[...]
    h1 = jnp.maximum(h1, 0.0).reshape(_FB, 24, 24, 20)
    h1 = jnp.max(h1.reshape(_FB, 12, 2, 24, 20), axis=2)
    h1 = jnp.max(h1.reshape(_FB, 12, 12, 2, 20), axis=3)
    p = h1.reshape(_FB, 12, 240)
    xc2 = jnp.concatenate([p[:, dy:dy + 8, :] for dy in range(5)], axis=2)
    xc2 = xc2.reshape(_FB * 8, 1200)
    h2 = jax.lax.dot(xc2, m2_ref[...], precision=_HI) + b2_ref[...]
    h2 = jnp.maximum(h2, 0.0).reshape(_FB, 8, 8, 50)
    h2 = jnp.max(h2.reshape(_FB, 4, 2, 8, 50), axis=2)
    h2 = jnp.max(h2.reshape(_FB, 4, 4, 2, 50), axis=3)
    hf = h2.reshape(_FB, 800)
    out = jax.lax.dot(hf, fw_ref[...], precision=_HI) + fb_ref[...]
    h_ref[...] = jnp.maximum(out, 0.0)


def _dist_body(hb_ref, h_ref, dist_ref, cnt_ref):
    i = pl.program_id(0)
    Hb = hb_ref[...]
    H = h_ref[...]
    sqb = jnp.sum(Hb * Hb, axis=1, keepdims=True)
    sqa = jnp.sum(H * H, axis=1, keepdims=True).reshape(1, _N)
    G = jax.lax.dot_general(Hb, H, (((1,), (1,)), ((), ())), precision=_HI)
    dist = jnp.sqrt(jnp.maximum(sqb - 2.0 * G + sqa, 0.0))
    row = i * _DB + jax.lax.broadcasted_iota(jnp.int32, (_DB, _N), 0)
    col = jax.lax.broadcasted_iota(jnp.int32, (_DB, _N), 1)
    dist = jnp.where(row != col, dist, jnp.float32(jnp.inf))
    dist_ref[...] = dist
    cnt = jnp.concatenate(
        [jnp.sum(jnp.where(dist < 2.5 + 0.5 * j, 1.0, 0.0)).reshape(1, 1)
         for j in range(8)], axis=1)

    @pl.when(i == 0)
    def _():
        cnt_ref[...] = jnp.zeros_like(cnt_ref)

    cnt_ref[...] += cnt


def _count8_body(dist_ref, base_ref, cnt_ref):
    i = pl.program_id(0)
    dist = dist_ref[...]
    base = base_ref[0, 0]
    cnt = jnp.concatenate(
        [jnp.sum(jnp.where(dist < base + 0.5 * j, 1.0, 0.0)).reshape(1, 1)
         for j in range(8)], axis=1)

    @pl.when(i == 0)
    def _():
        cnt_ref[...] = jnp.zeros_like(cnt_ref)

    cnt_ref[...] += cnt


def _aggdeg_body(dist_ref, hb_ref, t_ref, u_ref, deg_ref):
    i = pl.program_id(0)
    af = jnp.where(dist_ref[...] < t_ref[0, 0], 1.0, 0.0)

    @pl.when(i == 0)
    def _():
        u_ref[...] = jnp.zeros_like(u_ref)
        deg_ref[...] = jnp.zeros_like(deg_ref)

    u_ref[...] += jax.lax.dot_general(af, hb_ref[...],
                                      (((0,), (0,)), ((), ())),
                                      precision=_HI)
    deg_ref[...] += jnp.sum(af, axis=0, keepdims=True)


def _sage_bn_body(h_ref, u_ref, deg_ref, wl_ref, bl_ref, wr_ref,
                  bng_ref, bnb_ref, z_ref):
    aggr = u_ref[...] / jnp.transpose(jnp.maximum(deg_ref[...], 1.0))
    s1 = (jax.lax.dot(aggr, wl_ref[...], precision=_HI) + bl_ref[...]
          + jax.lax.dot(h_ref[...], wr_ref[...], precision=_HI))
    z_ref[...] = _bnorm(_lrelu(s1), bng_ref[...], bnb_ref[...])


def _clust_body(h_ref, u_ref, deg_ref, cwo_ref, cbo_ref, cwr_ref, s_ref):
    degsl = jnp.maximum(deg_ref[...] + 1.0, 1.0)
    prop = (u_ref[...] + h_ref[...]) / jnp.transpose(degsl)
    logits = (jax.lax.dot(prop, cwo_ref[...], precision=_HI) + cbo_ref[...]
              + jax.lax.dot(h_ref[...], cwr_ref[...], precision=_HI))
    m = jnp.max(logits, axis=1, keepdims=True)
    e = jnp.exp(logits - m)
    s_ref[...] = e / jnp.sum(e, axis=1, keepdims=True)


def _prop5_body(dist_ref, s_ref, t_ref, p_ref):
    af = jnp.where(dist_ref[...] < t_ref[0, 0], 1.0, 0.0)
    p_ref[...] = jax.lax.dot(af, s_ref[...], precision=_HI)


def _head2_body(z_ref, s_ref, p_ref, wl_ref, bl_ref, wr_ref,
                bng_ref, bnb_ref, l1w_ref, l1b_ref, l2w_ref, l2b_ref,
                out_ref):
    S = s_ref[...]
    coar = jax.lax.dot_general(S, z_ref[...], (((0,), (0,)), ((), ())),
                               precision=_HI)
    W2 = jax.lax.dot_general(S, p_ref[...], (((0,), (0,)), ((), ())),
                             precision=_HI)
    W2 = jnp.where(W2 < jnp.mean(W2), 0.0, W2)
    A2 = jnp.where(W2 != 0.0, 1.0, 0.0)

    deg2 = jnp.maximum(jnp.sum(A2, axis=0, keepdims=True), 1.0)
    aggr2 = jax.lax.dot_general(A2, coar, (((0,), (0,)), ((), ())),
                                precision=_HI) / jnp.transpose(deg2)
    s2 = (jax.lax.dot(aggr2, wl_ref[...], precision=_HI) + bl_ref[...]
          + jax.lax.dot(coar, wr_ref[...], precision=_HI))
    emb = _bnorm(_lrelu(s2), bng_ref[...], bnb_ref[...])

    r5 = jax.lax.broadcasted_iota(jnp.int32, (5, 5), 0)
    c5 = jax.lax.broadcasted_iota(jnp.int32, (5, 5), 1)
    asl = jnp.maximum(A2, jnp.where(r5 == c5, 1.0, 0.0))
    pooled = jnp.max(
        jnp.where(asl[:, :, None] > 0.0, emb[:, None, :],
                  jnp.float32(-jnp.inf)), axis=0)

    hh = _lrelu(jax.lax.dot(pooled, l1w_ref[...], precision=_HI)
                + l1b_ref[...])
    hh = _lrelu(jax.lax.dot(hh, l2w_ref[...], precision=_HI)
                + l2b_ref[...])
    mm = jnp.max(hh)
    ee = jnp.exp(hh - mm)
    prob = ee / jnp.sum(ee)
    out_ref[...] = jnp.max(prob).reshape(1, 1)


def _band1(w):
    wg = jnp.transpose(w[:, 0, :, :], (1, 2, 0))
    r = jnp.arange(140)
    c = jnp.arange(480)
    dy = (r // 28)[:, None]
    xx = (r % 28)[:, None]
    xpos = (c // 20)[None, :]
    o = (c % 20)[None, :]
    k = xx - xpos
    valid = (k >= 0) & (k < 5)
    kcl = jnp.clip(k, 0, 4)
    dyb = jnp.broadcast_to(dy, (140, 480))
    ob = jnp.broadcast_to(o, (140, 480))
    return jnp.where(valid, wg[dyb, kcl, ob], 0.0)


def _band2(w):
    wg = jnp.transpose(w, (2, 3, 1, 0))
    r = jnp.arange(1200)
    cc = jnp.arange(400)
    dy = (r // 240)[:, None]
    xx = ((r % 240) // 20)[:, None]
    ch = (r % 20)[:, None]
    xpos = (cc // 50)[None, :]
    o = (cc % 50)[None, :]
    k = xx - xpos
    valid = (k >= 0) & (k < 5)
    kcl = jnp.clip(k, 0, 4)
    dyb = jnp.broadcast_to(dy, (1200, 400))
    chb = jnp.broadcast_to(ch, (1200, 400))
    ob = jnp.broadcast_to(o, (1200, 400))
    return jnp.where(valid, wg[dyb, kcl, chb, ob], 0.0)


def kernel(x, conv1_w, conv1_b, conv2_w, conv2_b, fc_w, fc_b, sage_wl,
           sage_bl, sage_wr, bn_g, bn_b, clust_w_out, clust_b_out,
           clust_w_root, lin1_w, lin1_b, lin2_w, lin2_b):
    f32 = jnp.float32
    x3 = x.reshape(_N, 28, 28).astype(f32)

    m1 = _band1(conv1_w)
    b1t = jnp.tile(conv1_b, 24)[None, :]
    m2 = _band2(conv2_w)
    b2t = jnp.tile(conv2_b, 8)[None, :]
    fwp = jnp.transpose(fc_w.reshape(500, 50, 4, 4), (0, 2, 3, 1))
    fwp = fwp.reshape(500, 800).T

    nfb = _N // _FB
    H = pl.pallas_call(
        _feat_body,
        grid=(nfb,),
        in_specs=[
            pl.BlockSpec((_FB, 28, 28), lambda i: (i, 0, 0)),
            pl.BlockSpec((140, 480), lambda i: (0, 0)),
            pl.BlockSpec((1, 480), lambda i: (0, 0)),
            pl.BlockSpec((1200, 400), lambda i: (0, 0)),
            pl.BlockSpec((1, 400), lambda i: (0, 0)),
            pl.BlockSpec((800, 500), lambda i: (0, 0)),
            pl.BlockSpec((1, 500), lambda i: (0, 0)),
        ],
        out_specs=pl.BlockSpec((_FB, 500), lambda i: (i, 0)),
        out_shape=jax.ShapeDtypeStruct((_N, 500), f32),
    )(x3, m1, b1t, m2, b2t, fwp, fc_b[None, :])

    dist, cnt0 = pl.pallas_call(
        _dist_body,
        grid=(_NDB,),
        in_specs=[
            pl.BlockSpec((_DB, 500), lambda i: (i, 0)),
            pl.BlockSpec((_N, 500), lambda i: (0, 0)),
        ],
        out_specs=[
            pl.BlockSpec((_DB, _N), lambda i: (i, 0)),
            pl.BlockSpec((1, 8), lambda i: (0, 0)),
        ],
        out_shape=[
            jax.ShapeDtypeStruct((_N, _N), f32),
            jax.ShapeDtypeStruct((1, 8), f32),
        ],
    )(H, H)

    def _count8(base):
        return pl.pallas_call(
            _count8_body,
            grid=(_NDB,),
            in_specs=[
                pl.BlockSpec((_DB, _N), lambda i: (i, 0)),
                pl.BlockSpec((1, 1), lambda i: (0, 0)),
            ],
            out_specs=pl.BlockSpec((1, 8), lambda i: (0, 0)),
            out_shape=jax.ShapeDtypeStruct((1, 8), f32),
        )(dist, base.reshape(1, 1))

    thr = jnp.float32(_THRESH)

    def _w_cond(c):
        return c[1][0, 7] < thr

    def _w_body(c):
        nb = c[0] + 4.0
        return (nb, _count8(nb))

    base_f, cnt_f = jax.lax.while_loop(
        _w_cond, _w_body, (jnp.float32(2.5), cnt0))
    j = jnp.argmax(cnt_f[0, :] >= thr)
    t = (base_f + 0.5 * j.astype(f32)).reshape(1, 1)

    u, deg = pl.pallas_call(
        _aggdeg_body,
        grid=(_NDB,),
        in_specs=[
            pl.BlockSpec((_DB, _N), lambda i: (i, 0)),
            pl.BlockSpec((_DB, 500), lambda i: (i, 0)),
            pl.BlockSpec((1, 1), lambda i: (0, 0)),
        ],
        out_specs=[
            pl.BlockSpec((_N, 500), lambda i: (0, 0)),
            pl.BlockSpec((1, _N), lambda i: (0, 0)),
        ],
        out_shape=[
            jax.ShapeDtypeStruct((_N, 500), f32),
            jax.ShapeDtypeStruct((1, _N), f32),
        ],
    )(dist, H, t)

    z = pl.pallas_call(
        _sage_bn_body,
        out_shape=jax.ShapeDtypeStruct((_N, 500), f32),
        compiler_params=pltpu.CompilerParams(
            vmem_limit_bytes=64 * 1024 * 1024),
    )(H, u, deg, sage_wl.T, sage_bl[None, :], sage_wr.T,
      bn_g[None, :], bn_b[None, :])

    s = pl.pallas_call(
        _clust_body,
        out_shape=jax.ShapeDtypeStruct((_N, 5), f32),
        compiler_params=pltpu.CompilerParams(
            vmem_limit_bytes=64 * 1024 * 1024),
    )(H, u, deg, clust_w_out.T, clust_b_out[None, :], clust_w_root.T)

    p = pl.pallas_call(
        _prop5_body,
        grid=(_NDB,),
        in_specs=[
            pl.BlockSpec((_DB, _N), lambda i: (i, 0)),
            pl.BlockSpec((_N, 5), lambda i: (0, 0)),
            pl.BlockSpec((1, 1), lambda i: (0, 0)),
        ],
        out_specs=pl.BlockSpec((_DB, 5), lambda i: (i, 0)),
        out_shape=jax.ShapeDtypeStruct((_N, 5), f32),
    )(dist, s, t)

    out = pl.pallas_call(
        _head2_body,
        out_shape=jax.ShapeDtypeStruct((1, 1), f32),
        compiler_params=pltpu.CompilerParams(
            vmem_limit_bytes=64 * 1024 * 1024),
    )(z, s, p, sage_wl.T, sage_bl[None, :], sage_wr.T,
      bn_g[None, :], bn_b[None, :], lin1_w.T, lin1_b[None, :],
      lin2_w.T, lin2_b[None, :])

    y_prob = out[0, 0]
    y_hat = (y_prob >= 0.5).astype(f32)
    return (y_prob, y_hat)


# speedup vs baseline: 4.2692x; 4.2692x over previous
import jax
import jax.numpy as jnp
from jax.experimental import pallas as pl
from jax.experimental.pallas import tpu as pltpu

_HI = jax.lax.Precision.HIGHEST
_N = 2048
_FB = 128
_DB = 128
_NDB = _N // _DB
_THRESH = 0.1 * _N


def _lrelu(x):
    return jnp.where(x >= 0, x, 0.01 * x)


def _bnorm(x, g, b, eps=1e-5):
    mu = jnp.mean(x, axis=0, keepdims=True)
    var = jnp.mean((x - mu) * (x - mu), axis=0, keepdims=True)
    return (x - mu) / jnp.sqrt(var + eps) * g + b


def _feat_body(x_ref, w1_ref, b1_ref, w2_ref, b2_ref, fw_ref, fb_ref, h_ref):
    X = x_ref[...].reshape(28, 28, _FB)
    s = b1_ref[...].reshape(20, 1, 1, 1)
    s = jnp.broadcast_to(s, (20, 24, 24, _FB))
    for dy in range(5):
        for dx in range(5):
            xs = X[dy:dy + 24, dx:dx + 24, :]
            w = w1_ref[:, 5 * dy + dx:5 * dy + dx + 1].reshape(20, 1, 1, 1)
            s = s + w * xs[None, :, :, :]
    h1 = jnp.maximum(s, 0.0)
    h1 = jnp.max(h1.reshape(20, 12, 2, 24, _FB), axis=2)
    h1 = jnp.max(h1.reshape(20, 12, 12, 2, _FB), axis=3)
    im2 = jnp.concatenate(
        [h1[:, dy:dy + 8, dx:dx + 8, :].reshape(20, 64 * _FB)
         for dy in range(5) for dx in range(5)], axis=0)
    h2 = jax.lax.dot(w2_ref[...], im2, precision=_HI) + b2_ref[...]
    h2 = jnp.maximum(h2, 0.0).reshape(50, 8, 8, _FB)
    h2 = jnp.max(h2.reshape(50, 4, 2, 8, _FB), axis=2)
    h2 = jnp.max(h2.reshape(50, 4, 4, 2, _FB), axis=3)
    hf = h2.reshape(800, _FB)
    out = jax.lax.dot(fw_ref[...], hf, precision=_HI) + fb_ref[...]
    h_ref[...] = jnp.maximum(out, 0.0)


def _dist_body(hb_ref, h_ref, dist_ref, cnt_ref):
    i = pl.program_id(0)
    Hb = hb_ref[...]
    H = h_ref[...]
    sqb = jnp.sum(Hb * Hb, axis=1, keepdims=True)
    sqa = jnp.sum(H * H, axis=1, keepdims=True).reshape(1, _N)
    G = jax.lax.dot_general(Hb, H, (((1,), (1,)), ((), ())), precision=_HI)
    dist = jnp.sqrt(jnp.maximum(sqb - 2.0 * G + sqa, 0.0))
    row = i * _DB + jax.lax.broadcasted_iota(jnp.int32, (_DB, _N), 0)
    col = jax.lax.broadcasted_iota(jnp.int32, (_DB, _N), 1)
    dist = jnp.where(row != col, dist, jnp.float32(jnp.inf))
    dist_ref[...] = dist
    cnt = jnp.concatenate(
        [jnp.sum(jnp.where(dist < 2.5 + 0.5 * j, 1.0, 0.0)).reshape(1, 1)
         for j in range(8)], axis=1)

    @pl.when(i == 0)
    def _():
        cnt_ref[...] = jnp.zeros_like(cnt_ref)

    cnt_ref[...] += cnt


def _count8_body(dist_ref, base_ref, cnt_ref):
    i = pl.program_id(0)
    dist = dist_ref[...]
    base = base_ref[0, 0]
    cnt = jnp.concatenate(
        [jnp.sum(jnp.where(dist < base + 0.5 * j, 1.0, 0.0)).reshape(1, 1)
         for j in range(8)], axis=1)

    @pl.when(i == 0)
    def _():
        cnt_ref[...] = jnp.zeros_like(cnt_ref)

    cnt_ref[...] += cnt


def _aggdeg_body(dist_ref, hb_ref, t_ref, u_ref, deg_ref):
    i = pl.program_id(0)
    af = jnp.where(dist_ref[...] < t_ref[0, 0], 1.0, 0.0)

    @pl.when(i == 0)
    def _():
        u_ref[...] = jnp.zeros_like(u_ref)
        deg_ref[...] = jnp.zeros_like(deg_ref)

    u_ref[...] += jax.lax.dot_general(af, hb_ref[...],
                                      (((0,), (0,)), ((), ())),
                                      precision=_HI)
    deg_ref[...] += jnp.sum(af, axis=0, keepdims=True)


def _sage_bn_body(h_ref, u_ref, deg_ref, wl_ref, bl_ref, wr_ref,
                  bng_ref, bnb_ref, z_ref):
    aggr = u_ref[...] / jnp.transpose(jnp.maximum(deg_ref[...], 1.0))
    s1 = (jax.lax.dot(aggr, wl_ref[...], precision=_HI) + bl_ref[...]
          + jax.lax.dot(h_ref[...], wr_ref[...], precision=_HI))
    z_ref[...] = _bnorm(_lrelu(s1), bng_ref[...], bnb_ref[...])


def _clust_body(h_ref, u_ref, deg_ref, cwo_ref, cbo_ref, cwr_ref, s_ref):
    degsl = jnp.maximum(deg_ref[...] + 1.0, 1.0)
    prop = (u_ref[...] + h_ref[...]) / jnp.transpose(degsl)
    logits = (jax.lax.dot(prop, cwo_ref[...], precision=_HI) + cbo_ref[...]
              + jax.lax.dot(h_ref[...], cwr_ref[...], precision=_HI))
    m = jnp.max(logits, axis=1, keepdims=True)
    e = jnp.exp(logits - m)
    s_ref[...] = e / jnp.sum(e, axis=1, keepdims=True)


def _prop5_body(dist_ref, s_ref, t_ref, p_ref):
    af = jnp.where(dist_ref[...] < t_ref[0, 0], 1.0, 0.0)
    p_ref[...] = jax.lax.dot(af, s_ref[...], precision=_HI)


def _head2_body(z_ref, s_ref, p_ref, wl_ref, bl_ref, wr_ref,
                bng_ref, bnb_ref, l1w_ref, l1b_ref, l2w_ref, l2b_ref,
                out_ref):
    S = s_ref[...]
    coar = jax.lax.dot_general(S, z_ref[...], (((0,), (0,)), ((), ())),
                               precision=_HI)
    W2 = jax.lax.dot_general(S, p_ref[...], (((0,), (0,)), ((), ())),
                             precision=_HI)
    W2 = jnp.where(W2 < jnp.mean(W2), 0.0, W2)
    A2 = jnp.where(W2 != 0.0, 1.0, 0.0)

    deg2 = jnp.maximum(jnp.sum(A2, axis=0, keepdims=True), 1.0)
    aggr2 = jax.lax.dot_general(A2, coar, (((0,), (0,)), ((), ())),
                                precision=_HI) / jnp.transpose(deg2)
    s2 = (jax.lax.dot(aggr2, wl_ref[...], precision=_HI) + bl_ref[...]
          + jax.lax.dot(coar, wr_ref[...], precision=_HI))
    emb = _bnorm(_lrelu(s2), bng_ref[...], bnb_ref[...])

    r5 = jax.lax.broadcasted_iota(jnp.int32, (5, 5), 0)
    c5 = jax.lax.broadcasted_iota(jnp.int32, (5, 5), 1)
    asl = jnp.maximum(A2, jnp.where(r5 == c5, 1.0, 0.0))
    pooled = jnp.max(
        jnp.where(asl[:, :, None] > 0.0, emb[:, None, :],
                  jnp.float32(-jnp.inf)), axis=0)

    hh = _lrelu(jax.lax.dot(pooled, l1w_ref[...], precision=_HI)
                + l1b_ref[...])
    hh = _lrelu(jax.lax.dot(hh, l2w_ref[...], precision=_HI)
                + l2b_ref[...])
    mm = jnp.max(hh)
    ee = jnp.exp(hh - mm)
    prob = ee / jnp.sum(ee)
    out_ref[...] = jnp.max(prob).reshape(1, 1)


def kernel(x, conv1_w, conv1_b, conv2_w, conv2_b, fc_w, fc_b, sage_wl,
           sage_bl, sage_wr, bn_g, bn_b, clust_w_out, clust_b_out,
           clust_w_root, lin1_w, lin1_b, lin2_w, lin2_b):
    f32 = jnp.float32
    x3t = x.reshape(_N, 784).T.astype(f32)

    w1p = conv1_w.reshape(20, 25)
    w2p = jnp.transpose(conv2_w, (0, 2, 3, 1)).reshape(50, 500)

    nfb = _N // _FB
    Ht = pl.pallas_call(
        _feat_body,
        grid=(nfb,),
        in_specs=[
            pl.BlockSpec((784, _FB), lambda i: (0, i)),
            pl.BlockSpec((20, 25), lambda i: (0, 0)),
            pl.BlockSpec((20, 1), lambda i: (0, 0)),
            pl.BlockSpec((50, 500), lambda i: (0, 0)),
            pl.BlockSpec((50, 1), lambda i: (0, 0)),
            pl.BlockSpec((500, 800), lambda i: (0, 0)),
            pl.BlockSpec((500, 1), lambda i: (0, 0)),
        ],
        out_specs=pl.BlockSpec((500, _FB), lambda i: (0, i)),
        out_shape=jax.ShapeDtypeStruct((500, _N), f32),
    )(x3t, w1p, conv1_b[:, None], w2p, conv2_b[:, None],
      fc_w, fc_b[:, None])
    H = Ht.T

    dist, cnt0 = pl.pallas_call(
        _dist_body,
        grid=(_NDB,),
        in_specs=[
            pl.BlockSpec((_DB, 500), lambda i: (i, 0)),
            pl.BlockSpec((_N, 500), lambda i: (0, 0)),
        ],
        out_specs=[
            pl.BlockSpec((_DB, _N), lambda i: (i, 0)),
            pl.BlockSpec((1, 8), lambda i: (0, 0)),
        ],
        out_shape=[
            jax.ShapeDtypeStruct((_N, _N), f32),
            jax.ShapeDtypeStruct((1, 8), f32),
        ],
    )(H, H)

    def _count8(base):
        return pl.pallas_call(
            _count8_body,
            grid=(_NDB,),
            in_specs=[
                pl.BlockSpec((_DB, _N), lambda i: (i, 0)),
                pl.BlockSpec((1, 1), lambda i: (0, 0)),
            ],
            out_specs=pl.BlockSpec((1, 8), lambda i: (0, 0)),
            out_shape=jax.ShapeDtypeStruct((1, 8), f32),
        )(dist, base.reshape(1, 1))

    thr = jnp.float32(_THRESH)

    def _w_cond(c):
        return c[1][0, 7] < thr

    def _w_body(c):
        nb = c[0] + 4.0
        return (nb, _count8(nb))

    base_f, cnt_f = jax.lax.while_loop(
        _w_cond, _w_body, (jnp.float32(2.5), cnt0))
    j = jnp.argmax(cnt_f[0, :] >= thr)
    t = (base_f + 0.5 * j.astype(f32)).reshape(1, 1)

    u, deg = pl.pallas_call(
        _aggdeg_body,
        grid=(_NDB,),
        in_specs=[
            pl.BlockSpec((_DB, _N), lambda i: (i, 0)),
            pl.BlockSpec((_DB, 500), lambda i: (i, 0)),
            pl.BlockSpec((1, 1), lambda i: (0, 0)),
        ],
        out_specs=[
            pl.BlockSpec((_N, 500), lambda i: (0, 0)),
            pl.BlockSpec((1, _N), lambda i: (0, 0)),
        ],
        out_shape=[
            jax.ShapeDtypeStruct((_N, 500), f32),
            jax.ShapeDtypeStruct((1, _N), f32),
        ],
    )(dist, H, t)

    z = pl.pallas_call(
        _sage_bn_body,
        out_shape=jax.ShapeDtypeStruct((_N, 500), f32),
        compiler_params=pltpu.CompilerParams(
            vmem_limit_bytes=64 * 1024 * 1024),
    )(H, u, deg, sage_wl.T, sage_bl[None, :], sage_wr.T,
      bn_g[None, :], bn_b[None, :])

    s = pl.pallas_call(
        _clust_body,
        out_shape=jax.ShapeDtypeStruct((_N, 5), f32),
        compiler_params=pltpu.CompilerParams(
            vmem_limit_bytes=64 * 1024 * 1024),
    )(H, u, deg, clust_w_out.T, clust_b_out[None, :], clust_w_root.T)

    p = pl.pallas_call(
        _prop5_body,
        grid=(_NDB,),
        in_specs=[
            pl.BlockSpec((_DB, _N), lambda i: (i, 0)),
            pl.BlockSpec((_N, 5), lambda i: (0, 0)),
            pl.BlockSpec((1, 1), lambda i: (0, 0)),
        ],
        out_specs=pl.BlockSpec((_DB, 5), lambda i: (i, 0)),
        out_shape=jax.ShapeDtypeStruct((_N, 5), f32),
    )(dist, s, t)

    out = pl.pallas_call(
        _head2_body,
        out_shape=jax.ShapeDtypeStruct((1, 1), f32),
        compiler_params=pltpu.CompilerParams(
            vmem_limit_bytes=64 * 1024 * 1024),
    )(z, s, p, sage_wl.T, sage_bl[None, :], sage_wr.T,
      bn_g[None, :], bn_b[None, :], lin1_w.T, lin1_b[None, :],
      lin2_w.T, lin2_b[None, :])

    y_prob = out[0, 0]
    y_hat = (y_prob >= 0.5).astype(f32)
    return (y_prob, y_hat)
